# baseline (speedup 1.0000x reference)
import jax
import jax.numpy as jnp
from jax import lax
from jax.experimental import pallas as pl
from jax.experimental.pallas import tpu as pltpu
from jax.experimental.pallas import tpu_sc as plsc

NC = 2
NS = 16
L = 16
NW = NC * NS
BATCH = 16384
EMBED = 32
B_PER_W = BATCH // NW
CHUNK = 128


def _mf_body(users_hbm, items_hbm, wu_hbm, wi_hbm, out_hbm,
             idx_u, idx_i, rows_u, rows_i, out_v, sem):
    wid = lax.axis_index("s") * NC + lax.axis_index("c")
    base = wid * B_PER_W

    pltpu.sync_copy(users_hbm.at[pl.ds(base, B_PER_W)], idx_u)
    pltpu.sync_copy(items_hbm.at[pl.ds(base, B_PER_W)], idx_i)

    copies = []
    for k in range(0, B_PER_W, CHUNK):
        copies.append(pltpu.async_copy(
            wu_hbm.at[idx_u.at[pl.ds(k, CHUNK)]], rows_u.at[pl.ds(k, CHUNK)], sem))
        copies.append(pltpu.async_copy(
            wi_hbm.at[idx_i.at[pl.ds(k, CHUNK)]], rows_i.at[pl.ds(k, CHUNK)], sem))
    for c in copies:
        c.wait()

    def block_body(blk, carry):
        row0 = blk * L
        rows16 = lax.iota(jnp.int32, L) + row0
        acc = jnp.zeros((L,), jnp.float32)
        for d in range(EMBED):
            col = jnp.full((L,), d, jnp.int32)
            cu = plsc.load_gather(rows_u, [rows16, col])
            ci = plsc.load_gather(rows_i, [rows16, col])
            acc = acc + cu * ci
        out_v[pl.ds(row0, L)] = acc
        return carry

    lax.fori_loop(0, B_PER_W // L, block_body, 0)

    pltpu.sync_copy(out_v, out_hbm.at[pl.ds(base, B_PER_W)])


def kernel(users, items, W_user, W_item):
    users = users.astype(jnp.int32)
    items = items.astype(jnp.int32)
    mesh = plsc.VectorSubcoreMesh(
        core_axis_name="c", subcore_axis_name="s",
        num_cores=NC, num_subcores=NS)
    f = pl.kernel(
        _mf_body,
        out_type=jax.ShapeDtypeStruct((BATCH,), jnp.float32),
        mesh=mesh,
        compiler_params=pltpu.CompilerParams(
            needs_layout_passes=False, use_tc_tiling_on_sc=False),
        scratch_types=[
            pltpu.VMEM((B_PER_W,), jnp.int32),
            pltpu.VMEM((B_PER_W,), jnp.int32),
            pltpu.VMEM((B_PER_W, EMBED), jnp.float32),
            pltpu.VMEM((B_PER_W, EMBED), jnp.float32),
            pltpu.VMEM((B_PER_W,), jnp.float32),
            pltpu.SemaphoreType.DMA,
        ],
    )
    return f(users, items, W_user, W_item)
